# baseline (speedup 1.0000x reference)
import dataclasses
import functools

import jax
import jax.numpy as jnp
from jax import lax
from jax.experimental import pallas as pl
from jax.experimental.pallas import tpu as pltpu
from jax.experimental.pallas import tpu_sc as plsc

N_NODES = 10000
D = 256
E = 160000
NC = 2
NS = 16
NW = NC * NS
B_PER_W = E // NW
W = 104
NFULL = B_PER_W // W
TAIL = B_PER_W - NFULL * W
LANES = 16
BLANES = 32


def _dot_kernel(table_hbm, ei_hbm, out_hbm,
                sidx_v, didx_v, ar0, br0, ar1, br1, ar2, br2, outv,
                sa0, sb0, sa1, sb1, sa2, sb2):
    wid = lax.axis_index("s") * NC + lax.axis_index("c")
    base = wid * B_PER_W
    pltpu.sync_copy(ei_hbm.at[pl.ds(base, B_PER_W)], sidx_v)
    pltpu.sync_copy(ei_hbm.at[pl.ds(E + base, B_PER_W)], didx_v)

    lane = lax.iota(jnp.int32, LANES)
    last_lane = jnp.full((LANES,), LANES - 1, jnp.int32)

    def _edge_dot(arows, brows, w):
        pair = 2
        acc_lo = acc_hi = None
        for c0 in range(0, D // BLANES, pair):
            psum = None
            for c in range(c0, c0 + pair):
                av = plsc.bitcast(arows[w, pl.ds(c * LANES, LANES)],
                                  jnp.bfloat16)
                bv = plsc.bitcast(brows[w, pl.ds(c * LANES, LANES)],
                                  jnp.bfloat16)
                prod = av * bv
                psum = prod if psum is None else psum + prod
            lo, hi = plsc.unpack(psum, format=plsc.PackFormat.INTERLEAVED)
            acc_lo = lo if acc_lo is None else acc_lo + lo
            acc_hi = hi if acc_hi is None else acc_hi + hi
        cs = jnp.cumsum(acc_lo + acc_hi)
        return lax.gather(
            cs, last_lane[:, None],
            lax.GatherDimensionNumbers(offset_dims=(),
                                       collapsed_slice_dims=(0,),
                                       start_index_map=(0,)),
            slice_sizes=(1,),
            mode=lax.GatherScatterMode.PROMISE_IN_BOUNDS)

    def _issue(k, arows, brows, sem_a, sem_b):
        off = k * W
        cp_a = pltpu.async_copy(
            table_hbm.at[sidx_v.at[pl.ds(off, W)]], arows, sem_a)
        cp_b = pltpu.async_copy(
            table_hbm.at[didx_v.at[pl.ds(off, W)]], brows, sem_b)
        return cp_a, cp_b

    def _wait(arows, brows, sem_a, sem_b):
        pltpu.make_async_copy(
            table_hbm.at[sidx_v.at[pl.ds(0, W)]], arows, sem_a).wait()
        pltpu.make_async_copy(
            table_hbm.at[didx_v.at[pl.ds(0, W)]], brows, sem_b).wait()

    def _group_of(arows, brows, w0, n_edges):
        res = jnp.zeros((LANES,), jnp.float32)
        for j in range(n_edges):
            res = jnp.where(lane == j, _edge_dot(arows, brows, w0 + j), res)
        return res

    def _compute_resident(k, arows, brows):
        @pl.loop(0, W // LANES)
        def _group(g):
            outv[pl.ds(g * LANES, LANES)] = _group_of(
                arows, brows, g * LANES, LANES)

        if W % LANES:
            outv[pl.ds((W // LANES) * LANES, LANES)] = _group_of(
                arows, brows, (W // LANES) * LANES, W % LANES)

        pltpu.sync_copy(outv.at[pl.ds(0, W)],
                        out_hbm.at[pl.ds(base + k * W, W)])

    bufs = ((ar0, br0, sa0, sb0), (ar1, br1, sa1, sb1), (ar2, br2, sa2, sb2))

    _issue(0, *bufs[0])
    _issue(1, *bufs[1])

    @pl.loop(0, NFULL - 2, step=3)
    def _ring(k):
        _wait(*bufs[0])
        _issue(k + 2, *bufs[2])
        _compute_resident(k, bufs[0][0], bufs[0][1])

        _wait(*bufs[1])

        @pl.when(k + 3 < NFULL)
        def _():
            _issue(k + 3, *bufs[0])

        _compute_resident(k + 1, bufs[1][0], bufs[1][1])

        _wait(*bufs[2])

        @pl.when(k + 4 < NFULL)
        def _():
            _issue(k + 4, *bufs[1])

        _compute_resident(k + 2, bufs[2][0], bufs[2][1])

    if TAIL:
        toff = NFULL * W
        ta = ar0.at[pl.ds(0, TAIL)]
        tb = br0.at[pl.ds(0, TAIL)]
        pltpu.async_copy(
            table_hbm.at[sidx_v.at[pl.ds(toff, TAIL)]], ta, sa0).wait()
        pltpu.async_copy(
            table_hbm.at[didx_v.at[pl.ds(toff, TAIL)]], tb, sb0).wait()
        outv[pl.ds(0, LANES)] = _group_of(ar0, br0, 0, TAIL)
        pltpu.sync_copy(outv.at[pl.ds(0, TAIL)],
                        out_hbm.at[pl.ds(base + toff, TAIL)])


@jax.jit
def kernel(node_representations, edge_index):
    ei = edge_index.astype(jnp.int32).reshape(2 * E)
    lo = lax.bitcast_convert_type(
        node_representations[:, :D // 2].astype(jnp.bfloat16),
        jnp.uint16).astype(jnp.uint32)
    hi = lax.bitcast_convert_type(
        node_representations[:, D // 2:].astype(jnp.bfloat16),
        jnp.uint16).astype(jnp.uint32)
    table = lax.bitcast_convert_type(lo | (hi << 16), jnp.int32)

    mesh = plsc.VectorSubcoreMesh(core_axis_name="c", subcore_axis_name="s")
    cp = pltpu.CompilerParams()
    if "needs_layout_passes" in pltpu.CompilerParams.__dataclass_fields__:
        cp = dataclasses.replace(cp, needs_layout_passes=False)
    k = functools.partial(
        pl.kernel,
        mesh=mesh,
        compiler_params=cp,
        out_type=jax.ShapeDtypeStruct((E,), jnp.float32),
        scratch_types=[
            pltpu.VMEM((B_PER_W,), jnp.int32),
            pltpu.VMEM((B_PER_W,), jnp.int32),
            pltpu.VMEM((W, D // 2), jnp.int32),
            pltpu.VMEM((W, D // 2), jnp.int32),
            pltpu.VMEM((W, D // 2), jnp.int32),
            pltpu.VMEM((W, D // 2), jnp.int32),
            pltpu.VMEM((W, D // 2), jnp.int32),
            pltpu.VMEM((W, D // 2), jnp.int32),
            pltpu.VMEM((W + (-W) % LANES, ), jnp.float32),
            pltpu.SemaphoreType.DMA,
            pltpu.SemaphoreType.DMA,
            pltpu.SemaphoreType.DMA,
            pltpu.SemaphoreType.DMA,
            pltpu.SemaphoreType.DMA,
            pltpu.SemaphoreType.DMA,
        ],
    )(_dot_kernel)
    logits = k(table, ei)
    return logits.reshape(E, 1)


# speedup vs baseline: 1.2676x; 1.2676x over previous
import dataclasses
import functools

import jax
import jax.numpy as jnp
from jax import lax
from jax.experimental import pallas as pl
from jax.experimental.pallas import tpu as pltpu
from jax.experimental.pallas import tpu_sc as plsc

N_NODES = 10000
D = 256
E = 160000
NC = 2
NS = 16
NW = NC * NS
B_PER_W = E // NW
W = 200
NCHUNK = B_PER_W // W
LANES = 16
BLANES = 32
OUTP = W + (-W) % LANES


def _dot_kernel(table_hbm, ei_hbm, out_hbm,
                sidx_v, didx_v, ar0, br0, ar1, br1, outv0, outv1,
                sa0, sb0, sa1, sb1, so0, so1):
    wid = lax.axis_index("s") * NC + lax.axis_index("c")
    base = wid * B_PER_W
    pltpu.sync_copy(ei_hbm.at[pl.ds(base, B_PER_W)], sidx_v)
    pltpu.sync_copy(ei_hbm.at[pl.ds(E + base, B_PER_W)], didx_v)

    lane = lax.iota(jnp.int32, LANES)
    last_lane = jnp.full((LANES,), LANES - 1, jnp.int32)

    def _edge_dot(arows, brows, w):
        pair = 2
        acc_lo = acc_hi = None
        for c0 in range(0, D // BLANES, pair):
            psum = None
            for c in range(c0, c0 + pair):
                av = plsc.bitcast(arows[w, pl.ds(c * LANES, LANES)],
                                  jnp.bfloat16)
                bv = plsc.bitcast(brows[w, pl.ds(c * LANES, LANES)],
                                  jnp.bfloat16)
                prod = av * bv
                psum = prod if psum is None else psum + prod
            lo, hi = plsc.unpack(psum, format=plsc.PackFormat.INTERLEAVED)
            acc_lo = lo if acc_lo is None else acc_lo + lo
            acc_hi = hi if acc_hi is None else acc_hi + hi
        cs = jnp.cumsum(acc_lo + acc_hi)
        return lax.gather(
            cs, last_lane[:, None],
            lax.GatherDimensionNumbers(offset_dims=(),
                                       collapsed_slice_dims=(0,),
                                       start_index_map=(0,)),
            slice_sizes=(1,),
            mode=lax.GatherScatterMode.PROMISE_IN_BOUNDS)

    def _issue(k, arows, brows, sem_a, sem_b):
        off = k * W
        cp_a = pltpu.async_copy(
            table_hbm.at[sidx_v.at[pl.ds(off, W)]], arows, sem_a)
        cp_b = pltpu.async_copy(
            table_hbm.at[didx_v.at[pl.ds(off, W)]], brows, sem_b)
        return cp_a, cp_b

    def _group_of(arows, brows, w0, n_edges):
        res = jnp.zeros((LANES,), jnp.float32)
        for j in range(n_edges):
            res = jnp.where(lane == j, _edge_dot(arows, brows, w0 + j), res)
        return res

    def _wait_out(outv, sem_o):
        pltpu.make_async_copy(outv.at[pl.ds(0, W)],
                              out_hbm.at[pl.ds(base, W)], sem_o).wait()

    def _compute_resident(k, arows, brows, outv, sem_o, wait_prev):
        if wait_prev is not None:
            @pl.when(wait_prev)
            def _():
                _wait_out(outv, sem_o)

        @pl.loop(0, W // LANES)
        def _group(g):
            outv[pl.ds(g * LANES, LANES)] = _group_of(
                arows, brows, g * LANES, LANES)

        if W % LANES:
            outv[pl.ds((W // LANES) * LANES, LANES)] = _group_of(
                arows, brows, (W // LANES) * LANES, W % LANES)

        pltpu.async_copy(outv.at[pl.ds(0, W)],
                         out_hbm.at[pl.ds(base + k * W, W)], sem_o)

    cp_a, cp_b = _issue(0, ar0, br0, sa0, sb0)
    cp_a.wait()
    cp_b.wait()

    @pl.loop(0, NCHUNK - 1, step=2)
    def _pair(k):
        cp_a, cp_b = _issue(k + 1, ar1, br1, sa1, sb1)
        _compute_resident(k, ar0, br0, outv0, so0, k > 0)
        cp_a.wait()
        cp_b.wait()
        cp_a2, cp_b2 = _issue(k + 2, ar0, br0, sa0, sb0)
        _compute_resident(k + 1, ar1, br1, outv1, so1, k > 0)
        cp_a2.wait()
        cp_b2.wait()

    _wait_out(outv0, so0)
    _compute_resident(NCHUNK - 1, ar0, br0, outv0, so0, None)
    _wait_out(outv0, so0)
    _wait_out(outv1, so1)


@jax.jit
def kernel(node_representations, edge_index):
    ei = edge_index.astype(jnp.int32).reshape(2 * E)
    lo = lax.bitcast_convert_type(
        node_representations[:, :D // 2].astype(jnp.bfloat16),
        jnp.uint16).astype(jnp.uint32)
    hi = lax.bitcast_convert_type(
        node_representations[:, D // 2:].astype(jnp.bfloat16),
        jnp.uint16).astype(jnp.uint32)
    table = lax.bitcast_convert_type(lo | (hi << 16), jnp.int32)

    mesh = plsc.VectorSubcoreMesh(core_axis_name="c", subcore_axis_name="s")
    cp = pltpu.CompilerParams()
    if "needs_layout_passes" in pltpu.CompilerParams.__dataclass_fields__:
        cp = dataclasses.replace(cp, needs_layout_passes=False)
    k = functools.partial(
        pl.kernel,
        mesh=mesh,
        compiler_params=cp,
        out_type=jax.ShapeDtypeStruct((E,), jnp.float32),
        scratch_types=[
            pltpu.VMEM((B_PER_W,), jnp.int32),
            pltpu.VMEM((B_PER_W,), jnp.int32),
            pltpu.VMEM((W, D // 2), jnp.int32),
            pltpu.VMEM((W, D // 2), jnp.int32),
            pltpu.VMEM((W, D // 2), jnp.int32),
            pltpu.VMEM((W, D // 2), jnp.int32),
            pltpu.VMEM((OUTP,), jnp.float32),
            pltpu.VMEM((OUTP,), jnp.float32),
            pltpu.SemaphoreType.DMA,
            pltpu.SemaphoreType.DMA,
            pltpu.SemaphoreType.DMA,
            pltpu.SemaphoreType.DMA,
            pltpu.SemaphoreType.DMA,
            pltpu.SemaphoreType.DMA,
        ],
    )(_dot_kernel)
    logits = k(table, ei)
    return logits.reshape(E, 1)
